# baseline (speedup 1.0000x reference)
import functools

import jax
import jax.numpy as jnp
from jax import lax
from jax.experimental import pallas as pl
from jax.experimental.pallas import tpu as pltpu
from jax.experimental.pallas import tpu_sc as plsc

N = 10000
E = 160000
F_IN = 256
C = 128

NC = 2
NS = 16
NP = 10240
EB = 128
NB = 80
HALF = NB // 2
EP = NB * NS * EB
ROWS_PER_TILE = NP // NS


def _mesh():
    return plsc.VectorSubcoreMesh(
        core_axis_name="c", subcore_axis_name="s", num_cores=NC, num_subcores=NS
    )


@functools.cache
def _build_deg_kernel():
    @functools.partial(
        pl.kernel,
        out_type=jax.ShapeDtypeStruct((NP,), jnp.float32),
        mesh=_mesh(),
        scratch_types=[
            pltpu.VMEM((NB, EB), jnp.int32),
            pltpu.VMEM((EB,), jnp.float32),
            pltpu.VMEM_SHARED((NP,), jnp.float32),
        ],
    )
    def _deg_kernel(dst_hbm, zeros_hbm, deg_hbm, dst_all, ones_v, acc):
        c = lax.axis_index("c")
        s = lax.axis_index("s")

        @pl.when(c == 0)
        def _():
            for i in range(EB // 16):
                ones_v[pl.ds(i * 16, 16)] = jnp.full((16,), 1.0,
                                                     dtype=jnp.float32)
            pltpu.sync_copy(dst_hbm.at[s], dst_all)
            pltpu.sync_copy(
                zeros_hbm.at[pl.ds(s * ROWS_PER_TILE, ROWS_PER_TILE)],
                acc.at[pl.ds(s * ROWS_PER_TILE, ROWS_PER_TILE)],
            )

        plsc.subcore_barrier()

        @pl.when(c == 0)
        def _():
            def body(j, carry):
                pltpu.sync_copy(ones_v, acc.at[dst_all.at[j]], add=True)
                return carry

            lax.fori_loop(0, NB, body, 0)

        plsc.subcore_barrier()

        @pl.when(c == 0)
        def _():
            pltpu.sync_copy(
                acc.at[pl.ds(s * ROWS_PER_TILE, ROWS_PER_TILE)],
                deg_hbm.at[pl.ds(s * ROWS_PER_TILE, ROWS_PER_TILE)],
            )

    return _deg_kernel


@functools.cache
def _build_hop_kernel():
    @functools.partial(
        pl.kernel,
        out_type=jax.ShapeDtypeStruct((NP, C), jnp.float32),
        mesh=_mesh(),
        scratch_types=[
            pltpu.VMEM((HALF, EB), jnp.int32),
            pltpu.VMEM((HALF, EB), jnp.int32),
            pltpu.VMEM((EB, C), jnp.float32),
            pltpu.VMEM((EB, C), jnp.float32),
            pltpu.VMEM_SHARED((NP, C), jnp.float32),
            pltpu.SemaphoreType.DMA,
            pltpu.SemaphoreType.DMA,
        ],
    )
    def _hop_kernel(u_hbm, src_hbm, dst_hbm, zrows_hbm, out_hbm,
                    src_all, dst_all, rows0, rows1, acc, sem0, sem1):
        rows = (rows0, rows1)
        sems = (sem0, sem1)
        c = lax.axis_index("c")
        s = lax.axis_index("s")

        @pl.when(c == 0)
        def _():
            pltpu.sync_copy(
                zrows_hbm.at[pl.ds(s * ROWS_PER_TILE, ROWS_PER_TILE)],
                acc.at[pl.ds(s * ROWS_PER_TILE, ROWS_PER_TILE)],
            )

        plsc.subcore_barrier()

        @pl.when(c == 0)
        def _():
            for half in range(2):
                pltpu.sync_copy(src_hbm.at[s, pl.ds(half * HALF, HALF)],
                                src_all)
                pltpu.sync_copy(dst_hbm.at[s, pl.ds(half * HALF, HALF)],
                                dst_all)
                pltpu.async_copy(u_hbm.at[src_all.at[0]], rows[0], sems[0])

                def step(base, carry):
                    for b in range(2):
                        j = base + b
                        nxt = 1 - b

                        @pl.when(j + 1 < HALF)
                        def _():
                            pltpu.async_copy(u_hbm.at[src_all.at[j + 1]],
                                             rows[nxt], sems[nxt])

                        pltpu.make_async_copy(u_hbm.at[src_all.at[j]],
                                              rows[b], sems[b]).wait()
                        pltpu.sync_copy(rows[b], acc.at[dst_all.at[j]],
                                        add=True)
                    return carry

                lax.fori_loop(0, HALF // 2, lambda i, cr: step(i * 2, cr), 0)

        plsc.subcore_barrier()

        @pl.when(c == 0)
        def _():
            pltpu.sync_copy(
                acc.at[pl.ds(s * ROWS_PER_TILE, ROWS_PER_TILE)],
                out_hbm.at[pl.ds(s * ROWS_PER_TILE, ROWS_PER_TILE)],
            )

    return _hop_kernel


_BLK = 256
_NBLK = NP // _BLK


def _mm_body(x_ref, w_ref, deg_ref, u1_ref):
    dinv = lax.rsqrt(deg_ref[...] + 1.0)
    z = lax.dot_general(x_ref[...], w_ref[...],
                        (((1,), (1,)), ((), ())),
                        preferred_element_type=jnp.float32)
    u1_ref[...] = z * dinv[:, None]


def _mid_body(p_ref, u1_ref, deg_ref, u2_ref):
    dinv = lax.rsqrt(deg_ref[...] + 1.0)
    t = p_ref[...] + u1_ref[...]
    u2_ref[...] = t * (dinv * dinv)[:, None]


def _fin_body(q_ref, u2_ref, deg_ref, b_ref, o_ref):
    dinv = lax.rsqrt(deg_ref[...] + 1.0)
    t = q_ref[...] + u2_ref[...]
    logits = t * dinv[:, None] + b_ref[...][None, :]
    m = jnp.max(logits, axis=1, keepdims=True)
    sh = logits - m
    lse = jnp.log(jnp.sum(jnp.exp(sh), axis=1, keepdims=True))
    o_ref[...] = sh - lse


def kernel(x, edge_index, W, b):
    x = x.astype(jnp.float32)
    W = W.astype(jnp.float32)
    b = b.astype(jnp.float32)
    src = edge_index[0].astype(jnp.int32)
    dst = edge_index[1].astype(jnp.int32)

    xp = jnp.pad(x, ((0, NP - N), (0, 0)))
    pad_e = EP - E
    src_p = jnp.concatenate([src, jnp.zeros((pad_e,), jnp.int32)])
    trash = N + (jnp.arange(pad_e, dtype=jnp.int32) % (NP - N))
    dst_p = jnp.concatenate([dst, trash])
    src3 = src_p.reshape(NS, NB, EB)
    dst3 = dst_p.reshape(NS, NB, EB)

    zeros1 = jnp.zeros((NP,), jnp.float32)
    zrows = jnp.zeros((NP, C), jnp.float32)

    deg = _build_deg_kernel()(dst3, zeros1)

    u1 = pl.pallas_call(
        _mm_body,
        grid=(_NBLK,),
        in_specs=[
            pl.BlockSpec((_BLK, F_IN), lambda i: (i, 0)),
            pl.BlockSpec((C, F_IN), lambda i: (0, 0)),
            pl.BlockSpec((_BLK,), lambda i: (i,)),
        ],
        out_specs=pl.BlockSpec((_BLK, C), lambda i: (i, 0)),
        out_shape=jax.ShapeDtypeStruct((NP, C), jnp.float32),
    )(xp, W, deg)

    p = _build_hop_kernel()(u1, src3, dst3, zrows)

    u2 = pl.pallas_call(
        _mid_body,
        grid=(_NBLK,),
        in_specs=[
            pl.BlockSpec((_BLK, C), lambda i: (i, 0)),
            pl.BlockSpec((_BLK, C), lambda i: (i, 0)),
            pl.BlockSpec((_BLK,), lambda i: (i,)),
        ],
        out_specs=pl.BlockSpec((_BLK, C), lambda i: (i, 0)),
        out_shape=jax.ShapeDtypeStruct((NP, C), jnp.float32),
    )(p, u1, deg)

    q = _build_hop_kernel()(u2, src3, dst3, zrows)

    out = pl.pallas_call(
        _fin_body,
        grid=(_NBLK,),
        in_specs=[
            pl.BlockSpec((_BLK, C), lambda i: (i, 0)),
            pl.BlockSpec((_BLK, C), lambda i: (i, 0)),
            pl.BlockSpec((_BLK,), lambda i: (i,)),
            pl.BlockSpec((C,), lambda i: (0,)),
        ],
        out_specs=pl.BlockSpec((_BLK, C), lambda i: (i, 0)),
        out_shape=jax.ShapeDtypeStruct((NP, C), jnp.float32),
    )(q, u2, deg, b)

    return out[:N]


# speedup vs baseline: 1.0581x; 1.0581x over previous
import functools

import jax
import jax.numpy as jnp
from jax import lax
from jax.experimental import pallas as pl
from jax.experimental.pallas import tpu as pltpu
from jax.experimental.pallas import tpu_sc as plsc

N = 10000
E = 160000
F_IN = 256
C = 128

NC = 2
NS = 16
NP = 10240
EB = 80
NB = 128
EPT = NB * EB
EP = NS * EPT
ROWS_PER_TILE = NP // NS


def _mesh():
    return plsc.VectorSubcoreMesh(
        core_axis_name="c", subcore_axis_name="s", num_cores=NC, num_subcores=NS
    )


@functools.cache
def _build_deg_kernel():
    @functools.partial(
        pl.kernel,
        out_type=jax.ShapeDtypeStruct((NP,), jnp.float32),
        mesh=_mesh(),
        scratch_types=[
            pltpu.VMEM((NB, EB), jnp.int32),
            pltpu.VMEM((EB,), jnp.float32),
            pltpu.VMEM_SHARED((NP,), jnp.float32),
        ],
    )
    def _deg_kernel(dst_hbm, zeros_hbm, deg_hbm, dst_all, ones_v, acc):
        c = lax.axis_index("c")
        s = lax.axis_index("s")

        @pl.when(c == 0)
        def _():
            for i in range(EB // 16):
                ones_v[pl.ds(i * 16, 16)] = jnp.full((16,), 1.0,
                                                     dtype=jnp.float32)
            pltpu.sync_copy(dst_hbm.at[s], dst_all)
            pltpu.sync_copy(
                zeros_hbm.at[pl.ds(s * ROWS_PER_TILE, ROWS_PER_TILE)],
                acc.at[pl.ds(s * ROWS_PER_TILE, ROWS_PER_TILE)],
            )

        plsc.subcore_barrier()

        @pl.when(c == 0)
        def _():
            def body(j, carry):
                pltpu.sync_copy(ones_v, acc.at[dst_all.at[j]], add=True)
                return carry

            lax.fori_loop(0, NB, body, 0)

        plsc.subcore_barrier()

        @pl.when(c == 0)
        def _():
            pltpu.sync_copy(
                acc.at[pl.ds(s * ROWS_PER_TILE, ROWS_PER_TILE)],
                deg_hbm.at[pl.ds(s * ROWS_PER_TILE, ROWS_PER_TILE)],
            )

    return _deg_kernel


@functools.cache
def _build_hop_kernel():
    @functools.partial(
        pl.kernel,
        out_type=jax.ShapeDtypeStruct((NP, C), jnp.float32),
        mesh=_mesh(),
        scratch_types=[
            pltpu.VMEM((EPT,), jnp.int32),
            pltpu.VMEM((NB, EB), jnp.int32),
            pltpu.VMEM((EB, C), jnp.float32),
            pltpu.VMEM((EB, C), jnp.float32),
            pltpu.VMEM_SHARED((NP, C), jnp.float32),
            pltpu.SemaphoreType.DMA,
            pltpu.SemaphoreType.DMA,
        ],
    )
    def _hop_kernel(u_hbm, src_hbm, dst_hbm, zrows_hbm, out_hbm,
                    src_all, dst_all, rows0, rows1, acc, sem0, sem1):
        rows = (rows0, rows1)
        sems = (sem0, sem1)
        c = lax.axis_index("c")
        s = lax.axis_index("s")

        @pl.when(c == 0)
        def _():
            pltpu.sync_copy(src_hbm.at[s], src_all)
            pltpu.sync_copy(dst_hbm.at[s], dst_all)
            pltpu.sync_copy(
                zrows_hbm.at[pl.ds(s * ROWS_PER_TILE, ROWS_PER_TILE)],
                acc.at[pl.ds(s * ROWS_PER_TILE, ROWS_PER_TILE)],
            )

        plsc.subcore_barrier()

        @pl.when(c == 0)
        def _():
            pltpu.async_copy(u_hbm.at[src_all.at[pl.ds(0, EB)]], rows[0], sems[0])

            def step(base, carry):
                for b in range(2):
                    j = base + b
                    nxt = 1 - b

                    @pl.when(j + 1 < NB)
                    def _():
                        pltpu.async_copy(
                            u_hbm.at[src_all.at[pl.ds((j + 1) * EB, EB)]],
                            rows[nxt], sems[nxt])

                    pltpu.make_async_copy(
                        u_hbm.at[src_all.at[pl.ds(j * EB, EB)]],
                        rows[b], sems[b]).wait()
                    pltpu.sync_copy(rows[b], acc.at[dst_all.at[j]],
                                    add=True)
                return carry

            lax.fori_loop(0, NB // 2, lambda i, cr: step(i * 2, cr), 0)

        plsc.subcore_barrier()

        @pl.when(c == 0)
        def _():
            pltpu.sync_copy(
                acc.at[pl.ds(s * ROWS_PER_TILE, ROWS_PER_TILE)],
                out_hbm.at[pl.ds(s * ROWS_PER_TILE, ROWS_PER_TILE)],
            )

    return _hop_kernel


_BLK = 256
_NBLK = NP // _BLK


def _mm_body(x_ref, w_ref, deg_ref, u1_ref):
    dinv = lax.rsqrt(deg_ref[...] + 1.0)
    z = lax.dot_general(x_ref[...], w_ref[...],
                        (((1,), (1,)), ((), ())),
                        preferred_element_type=jnp.float32)
    u1_ref[...] = z * dinv[:, None]


def _mid_body(p_ref, u1_ref, deg_ref, u2_ref):
    dinv = lax.rsqrt(deg_ref[...] + 1.0)
    t = p_ref[...] + u1_ref[...]
    u2_ref[...] = t * (dinv * dinv)[:, None]


def _fin_body(q_ref, u2_ref, deg_ref, b_ref, o_ref):
    dinv = lax.rsqrt(deg_ref[...] + 1.0)
    t = q_ref[...] + u2_ref[...]
    logits = t * dinv[:, None] + b_ref[...][None, :]
    m = jnp.max(logits, axis=1, keepdims=True)
    sh = logits - m
    lse = jnp.log(jnp.sum(jnp.exp(sh), axis=1, keepdims=True))
    o_ref[...] = sh - lse


def kernel(x, edge_index, W, b):
    x = x.astype(jnp.float32)
    W = W.astype(jnp.float32)
    b = b.astype(jnp.float32)
    src = edge_index[0].astype(jnp.int32)
    dst = edge_index[1].astype(jnp.int32)

    xp = jnp.pad(x, ((0, NP - N), (0, 0)))
    pad_e = EP - E
    src_p = jnp.concatenate([src, jnp.full((pad_e,), N, jnp.int32)])
    dst_p = jnp.concatenate([dst, jnp.full((pad_e,), N + 16, jnp.int32)])
    src3 = src_p.reshape(NS, EPT)
    dst3 = dst_p.reshape(NS, NB, EB)

    zeros1 = jnp.zeros((NP,), jnp.float32)
    zrows = jnp.zeros((NP, C), jnp.float32)

    deg = _build_deg_kernel()(dst3, zeros1)

    u1 = pl.pallas_call(
        _mm_body,
        grid=(_NBLK,),
        in_specs=[
            pl.BlockSpec((_BLK, F_IN), lambda i: (i, 0)),
            pl.BlockSpec((C, F_IN), lambda i: (0, 0)),
            pl.BlockSpec((_BLK,), lambda i: (i,)),
        ],
        out_specs=pl.BlockSpec((_BLK, C), lambda i: (i, 0)),
        out_shape=jax.ShapeDtypeStruct((NP, C), jnp.float32),
    )(xp, W, deg)

    p = _build_hop_kernel()(u1, src3, dst3, zrows)

    u2 = pl.pallas_call(
        _mid_body,
        grid=(_NBLK,),
        in_specs=[
            pl.BlockSpec((_BLK, C), lambda i: (i, 0)),
            pl.BlockSpec((_BLK, C), lambda i: (i, 0)),
            pl.BlockSpec((_BLK,), lambda i: (i,)),
        ],
        out_specs=pl.BlockSpec((_BLK, C), lambda i: (i, 0)),
        out_shape=jax.ShapeDtypeStruct((NP, C), jnp.float32),
    )(p, u1, deg)

    q = _build_hop_kernel()(u2, src3, dst3, zrows)

    out = pl.pallas_call(
        _fin_body,
        grid=(_NBLK,),
        in_specs=[
            pl.BlockSpec((_BLK, C), lambda i: (i, 0)),
            pl.BlockSpec((_BLK, C), lambda i: (i, 0)),
            pl.BlockSpec((_BLK,), lambda i: (i,)),
            pl.BlockSpec((C,), lambda i: (0,)),
        ],
        out_specs=pl.BlockSpec((_BLK, C), lambda i: (i, 0)),
        out_shape=jax.ShapeDtypeStruct((NP, C), jnp.float32),
    )(q, u2, deg, b)

    return out[:N]


# speedup vs baseline: 1.0819x; 1.0224x over previous
import functools

import jax
import jax.numpy as jnp
from jax import lax
from jax.experimental import pallas as pl
from jax.experimental.pallas import tpu as pltpu
from jax.experimental.pallas import tpu_sc as plsc

N = 10000
E = 160000
F_IN = 256
C = 128

NC = 2
NS = 16
NP = 10240
EB = 128
NB = 80
BLK8 = 8
NBLK8 = NB // BLK8
EPT = NB * EB
EP = NS * EPT
ROWS_PER_TILE = NP // NS


def _mesh():
    return plsc.VectorSubcoreMesh(
        core_axis_name="c", subcore_axis_name="s", num_cores=NC, num_subcores=NS
    )


@functools.cache
def _build_deg_kernel():
    @functools.partial(
        pl.kernel,
        out_type=jax.ShapeDtypeStruct((NP,), jnp.float32),
        mesh=_mesh(),
        scratch_types=[
            pltpu.VMEM((NB, EB), jnp.int32),
            pltpu.VMEM((EB,), jnp.float32),
            pltpu.VMEM_SHARED((NP,), jnp.float32),
        ],
    )
    def _deg_kernel(dst_hbm, zeros_hbm, deg_hbm, dst_all, ones_v, acc):
        c = lax.axis_index("c")
        s = lax.axis_index("s")

        @pl.when(c == 0)
        def _():
            for i in range(EB // 16):
                ones_v[pl.ds(i * 16, 16)] = jnp.full((16,), 1.0,
                                                     dtype=jnp.float32)
            pltpu.sync_copy(dst_hbm.at[s], dst_all)
            pltpu.sync_copy(
                zeros_hbm.at[pl.ds(s * ROWS_PER_TILE, ROWS_PER_TILE)],
                acc.at[pl.ds(s * ROWS_PER_TILE, ROWS_PER_TILE)],
            )

        plsc.subcore_barrier()

        @pl.when(c == 0)
        def _():
            def body(j, carry):
                pltpu.sync_copy(ones_v, acc.at[dst_all.at[j]], add=True)
                return carry

            lax.fori_loop(0, NB, body, 0)

        plsc.subcore_barrier()

        @pl.when(c == 0)
        def _():
            pltpu.sync_copy(
                acc.at[pl.ds(s * ROWS_PER_TILE, ROWS_PER_TILE)],
                deg_hbm.at[pl.ds(s * ROWS_PER_TILE, ROWS_PER_TILE)],
            )

    return _deg_kernel


@functools.cache
def _build_hop_kernel():
    @functools.partial(
        pl.kernel,
        out_type=jax.ShapeDtypeStruct((NP, C), jnp.float32),
        mesh=_mesh(),
        scratch_types=[
            pltpu.VMEM((NB, EB), jnp.int32),
            pltpu.VMEM((2, BLK8, EB), jnp.int32),
            pltpu.VMEM((EB, C), jnp.float32),
            pltpu.VMEM((EB, C), jnp.float32),
            pltpu.VMEM_SHARED((NP, C), jnp.float32),
            pltpu.SemaphoreType.DMA,
            pltpu.SemaphoreType.DMA,
            pltpu.SemaphoreType.DMA,
            pltpu.SemaphoreType.DMA,
        ],
    )
    def _hop_kernel(u_hbm, src_hbm, dst_hbm, zrows_hbm, out_hbm,
                    src_all, dring, rows0, rows1, acc,
                    sem0, sem1, dsem0, dsem1):
        rows = (rows0, rows1)
        sems = (sem0, sem1)
        dsems = (dsem0, dsem1)
        c = lax.axis_index("c")
        s = lax.axis_index("s")

        @pl.when(c == 0)
        def _():
            pltpu.sync_copy(src_hbm.at[s], src_all)
            pltpu.sync_copy(dst_hbm.at[s, pl.ds(0, BLK8)], dring.at[0])
            pltpu.sync_copy(
                zrows_hbm.at[pl.ds(s * ROWS_PER_TILE, ROWS_PER_TILE)],
                acc.at[pl.ds(s * ROWS_PER_TILE, ROWS_PER_TILE)],
            )

        plsc.subcore_barrier()

        @pl.when(c == 0)
        def _():
            pltpu.async_copy(dst_hbm.at[s, pl.ds(BLK8, BLK8)],
                             dring.at[1], dsems[1])
            pltpu.async_copy(u_hbm.at[src_all.at[0]], rows[0], sems[0])

            def block2(k2, carry):
              for h in range(2):
                k = k2 * 2 + h

                @pl.when(k > 0)
                def _():
                    pltpu.make_async_copy(
                        dst_hbm.at[s, pl.ds(k * BLK8, BLK8)],
                        dring.at[h], dsems[h]).wait()

                for slot in range(BLK8):
                    b = slot % 2
                    nxt = 1 - b
                    j = k * BLK8 + slot

                    @pl.when(j + 1 < NB)
                    def _():
                        pltpu.async_copy(u_hbm.at[src_all.at[j + 1]],
                                         rows[nxt], sems[nxt])

                    pltpu.make_async_copy(u_hbm.at[src_all.at[j]],
                                          rows[b], sems[b]).wait()
                    pltpu.sync_copy(rows[b], acc.at[dring.at[h, slot]],
                                    add=True)

                @pl.when(k + 2 < NBLK8)
                def _():
                    pltpu.async_copy(
                        dst_hbm.at[s, pl.ds((k + 2) * BLK8, BLK8)],
                        dring.at[h], dsems[h])

              return carry

            lax.fori_loop(0, NBLK8 // 2, block2, 0)

        plsc.subcore_barrier()

        @pl.when(c == 0)
        def _():
            pltpu.sync_copy(
                acc.at[pl.ds(s * ROWS_PER_TILE, ROWS_PER_TILE)],
                out_hbm.at[pl.ds(s * ROWS_PER_TILE, ROWS_PER_TILE)],
            )

    return _hop_kernel


_BLK = 256
_NBLK = NP // _BLK


def _mm_body(x_ref, w_ref, deg_ref, u1_ref):
    dinv = lax.rsqrt(deg_ref[...] + 1.0)
    z = lax.dot_general(x_ref[...], w_ref[...],
                        (((1,), (1,)), ((), ())),
                        preferred_element_type=jnp.float32)
    u1_ref[...] = z * dinv[:, None]


def _mid_body(p_ref, u1_ref, deg_ref, u2_ref):
    dinv = lax.rsqrt(deg_ref[...] + 1.0)
    t = p_ref[...] + u1_ref[...]
    u2_ref[...] = t * (dinv * dinv)[:, None]


def _fin_body(q_ref, u2_ref, deg_ref, b_ref, o_ref):
    dinv = lax.rsqrt(deg_ref[...] + 1.0)
    t = q_ref[...] + u2_ref[...]
    logits = t * dinv[:, None] + b_ref[...][None, :]
    m = jnp.max(logits, axis=1, keepdims=True)
    sh = logits - m
    lse = jnp.log(jnp.sum(jnp.exp(sh), axis=1, keepdims=True))
    o_ref[...] = sh - lse


def kernel(x, edge_index, W, b):
    x = x.astype(jnp.float32)
    W = W.astype(jnp.float32)
    b = b.astype(jnp.float32)
    src = edge_index[0].astype(jnp.int32)
    dst = edge_index[1].astype(jnp.int32)

    xp = jnp.pad(x, ((0, NP - N), (0, 0)))
    pad_e = EP - E
    src_p = jnp.concatenate([src, jnp.full((pad_e,), N, jnp.int32)])
    dst_p = jnp.concatenate([dst, jnp.full((pad_e,), N + 16, jnp.int32)])
    src3 = src_p.reshape(NS, NB, EB)
    dst3 = dst_p.reshape(NS, NB, EB)

    zeros1 = jnp.zeros((NP,), jnp.float32)
    zrows = jnp.zeros((NP, C), jnp.float32)

    deg = _build_deg_kernel()(dst3, zeros1)

    u1 = pl.pallas_call(
        _mm_body,
        grid=(_NBLK,),
        in_specs=[
            pl.BlockSpec((_BLK, F_IN), lambda i: (i, 0)),
            pl.BlockSpec((C, F_IN), lambda i: (0, 0)),
            pl.BlockSpec((_BLK,), lambda i: (i,)),
        ],
        out_specs=pl.BlockSpec((_BLK, C), lambda i: (i, 0)),
        out_shape=jax.ShapeDtypeStruct((NP, C), jnp.float32),
    )(xp, W, deg)

    p = _build_hop_kernel()(u1, src3, dst3, zrows)

    u2 = pl.pallas_call(
        _mid_body,
        grid=(_NBLK,),
        in_specs=[
            pl.BlockSpec((_BLK, C), lambda i: (i, 0)),
            pl.BlockSpec((_BLK, C), lambda i: (i, 0)),
            pl.BlockSpec((_BLK,), lambda i: (i,)),
        ],
        out_specs=pl.BlockSpec((_BLK, C), lambda i: (i, 0)),
        out_shape=jax.ShapeDtypeStruct((NP, C), jnp.float32),
    )(p, u1, deg)

    q = _build_hop_kernel()(u2, src3, dst3, zrows)

    out = pl.pallas_call(
        _fin_body,
        grid=(_NBLK,),
        in_specs=[
            pl.BlockSpec((_BLK, C), lambda i: (i, 0)),
            pl.BlockSpec((_BLK, C), lambda i: (i, 0)),
            pl.BlockSpec((_BLK,), lambda i: (i,)),
            pl.BlockSpec((C,), lambda i: (0,)),
        ],
        out_specs=pl.BlockSpec((_BLK, C), lambda i: (i, 0)),
        out_shape=jax.ShapeDtypeStruct((NP, C), jnp.float32),
    )(q, u2, deg, b)

    return out[:N]


# speedup vs baseline: 2.3052x; 2.1308x over previous
import functools

import jax
import jax.numpy as jnp
from jax import lax
from jax.experimental import pallas as pl
from jax.experimental.pallas import tpu as pltpu
from jax.experimental.pallas import tpu_sc as plsc

N = 10000
E = 160000
F_IN = 256
C = 128

NC = 2
NS = 16
NP = 10240
EB = 128
NB = 80
BLK8 = 8
NBLK8 = NB // BLK8
EPT = NB * EB
EP = NS * EPT
ROWS_PER_TILE = NP // NS


def _mesh():
    return plsc.VectorSubcoreMesh(
        core_axis_name="c", subcore_axis_name="s", num_cores=NC, num_subcores=NS
    )


@functools.cache
def _build_deg_kernel():
    @functools.partial(
        pl.kernel,
        out_type=jax.ShapeDtypeStruct((NP,), jnp.float32),
        mesh=_mesh(),
        scratch_types=[
            pltpu.VMEM((NB, EB), jnp.int32),
            pltpu.VMEM((EB,), jnp.float32),
            pltpu.VMEM_SHARED((NP,), jnp.float32),
        ],
    )
    def _deg_kernel(dst_hbm, zeros_hbm, deg_hbm, dst_all, ones_v, acc):
        c = lax.axis_index("c")
        s = lax.axis_index("s")

        @pl.when(c == 0)
        def _():
            for i in range(EB // 16):
                ones_v[pl.ds(i * 16, 16)] = jnp.full((16,), 1.0,
                                                     dtype=jnp.float32)
            pltpu.sync_copy(dst_hbm.at[s], dst_all)
            pltpu.sync_copy(
                zeros_hbm.at[pl.ds(s * ROWS_PER_TILE, ROWS_PER_TILE)],
                acc.at[pl.ds(s * ROWS_PER_TILE, ROWS_PER_TILE)],
            )

        plsc.subcore_barrier()

        @pl.when(c == 0)
        def _():
            def body(j, carry):
                pltpu.sync_copy(ones_v, acc.at[dst_all.at[j]], add=True)
                return carry

            lax.fori_loop(0, NB, body, 0)

        plsc.subcore_barrier()

        @pl.when(c == 0)
        def _():
            pltpu.sync_copy(
                acc.at[pl.ds(s * ROWS_PER_TILE, ROWS_PER_TILE)],
                deg_hbm.at[pl.ds(s * ROWS_PER_TILE, ROWS_PER_TILE)],
            )

    return _deg_kernel


@functools.cache
def _build_hop_kernel():
    @functools.partial(
        pl.kernel,
        out_type=jax.ShapeDtypeStruct((NP, C), jnp.float32),
        mesh=_mesh(),
        scratch_types=[
            pltpu.VMEM((NB, EB), jnp.int32),
            pltpu.VMEM((2, BLK8, EB), jnp.int32),
            pltpu.VMEM((EB, C), jnp.float32),
            pltpu.VMEM((EB, C), jnp.float32),
            pltpu.VMEM_SHARED((NP, C), jnp.float32),
            pltpu.SemaphoreType.DMA,
            pltpu.SemaphoreType.DMA,
            pltpu.SemaphoreType.DMA,
            pltpu.SemaphoreType.DMA,
        ],
    )
    def _hop_kernel(u_hbm, src_hbm, dst_hbm, zrows_hbm, out_hbm,
                    src_all, dring, rows0, rows1, acc,
                    sem0, sem1, dsem0, dsem1):
        rows = (rows0, rows1)
        sems = (sem0, sem1)
        dsems = (dsem0, dsem1)
        c = lax.axis_index("c")
        s = lax.axis_index("s")

        @pl.when(c == 0)
        def _():
            pltpu.sync_copy(src_hbm.at[s], src_all)
            pltpu.sync_copy(dst_hbm.at[s, pl.ds(0, BLK8)], dring.at[0])
            pltpu.sync_copy(
                zrows_hbm.at[pl.ds(s * ROWS_PER_TILE, ROWS_PER_TILE)],
                acc.at[pl.ds(s * ROWS_PER_TILE, ROWS_PER_TILE)],
            )

        plsc.subcore_barrier()

        @pl.when(c == 0)
        def _():
            pltpu.async_copy(dst_hbm.at[s, pl.ds(BLK8, BLK8)],
                             dring.at[1], dsems[1])
            pltpu.async_copy(u_hbm.at[src_all.at[0]], rows[0], sems[0])

            def block2(k2, carry):
              for h in range(2):
                k = k2 * 2 + h

                @pl.when(k > 0)
                def _():
                    pltpu.make_async_copy(
                        dst_hbm.at[s, pl.ds(k * BLK8, BLK8)],
                        dring.at[h], dsems[h]).wait()

                for slot in range(BLK8):
                    b = slot % 2
                    nxt = 1 - b
                    j = k * BLK8 + slot

                    @pl.when(j + 1 < NB)
                    def _():
                        pltpu.async_copy(u_hbm.at[src_all.at[j + 1]],
                                         rows[nxt], sems[nxt])

                    pltpu.make_async_copy(u_hbm.at[src_all.at[j]],
                                          rows[b], sems[b]).wait()
                    pltpu.sync_copy(rows[b], acc.at[dring.at[h, slot]],
                                    add=True)

                @pl.when(k + 2 < NBLK8)
                def _():
                    pltpu.async_copy(
                        dst_hbm.at[s, pl.ds((k + 2) * BLK8, BLK8)],
                        dring.at[h], dsems[h])

              return carry

            lax.fori_loop(0, NBLK8 // 2, block2, 0)

        plsc.subcore_barrier()

        @pl.when(c == 0)
        def _():
            pltpu.sync_copy(
                acc.at[pl.ds(s * ROWS_PER_TILE, ROWS_PER_TILE)],
                out_hbm.at[pl.ds(s * ROWS_PER_TILE, ROWS_PER_TILE)],
            )

    return _hop_kernel


_BLK = 256
_NBLK = NP // _BLK


def _mm_body(x_ref, w_ref, deg_ref, u1_ref):
    dinv = lax.rsqrt(deg_ref[...] + 1.0)
    z = lax.dot_general(x_ref[...], w_ref[...],
                        (((1,), (1,)), ((), ())),
                        preferred_element_type=jnp.float32)
    u1_ref[...] = z * dinv[:, None]


def _mid_body(p_ref, u1_ref, deg_ref, u2_ref):
    dinv = lax.rsqrt(deg_ref[...] + 1.0)
    t = p_ref[...] + u1_ref[...]
    u2_ref[...] = t * (dinv * dinv)[:, None]


def _fin_body(q_ref, u2_ref, deg_ref, b_ref, o_ref):
    dinv = lax.rsqrt(deg_ref[...] + 1.0)
    t = q_ref[...] + u2_ref[...]
    logits = t * dinv[:, None] + b_ref[...][None, :]
    m = jnp.max(logits, axis=1, keepdims=True)
    sh = logits - m
    lse = jnp.log(jnp.sum(jnp.exp(sh), axis=1, keepdims=True))
    o_ref[...] = sh - lse


def kernel(x, edge_index, W, b):
    x = x.astype(jnp.float32)
    W = W.astype(jnp.float32)
    b = b.astype(jnp.float32)
    src = edge_index[0].astype(jnp.int32)
    dst = edge_index[1].astype(jnp.int32)

    xp = jnp.pad(x, ((0, NP - N), (0, 0)))
    pad_e = EP - E
    spread = N + (jnp.arange(pad_e, dtype=jnp.int32) % (NP - N))
    src_p = jnp.concatenate([src, spread])
    dst_p = jnp.concatenate([dst, spread])
    src3 = src_p.reshape(NS, NB, EB)
    dst3 = dst_p.reshape(NS, NB, EB)

    zeros1 = jnp.zeros((NP,), jnp.float32)
    zrows = jnp.zeros((NP, C), jnp.float32)

    deg = _build_deg_kernel()(dst3, zeros1)

    u1 = pl.pallas_call(
        _mm_body,
        grid=(_NBLK,),
        in_specs=[
            pl.BlockSpec((_BLK, F_IN), lambda i: (i, 0)),
            pl.BlockSpec((C, F_IN), lambda i: (0, 0)),
            pl.BlockSpec((_BLK,), lambda i: (i,)),
        ],
        out_specs=pl.BlockSpec((_BLK, C), lambda i: (i, 0)),
        out_shape=jax.ShapeDtypeStruct((NP, C), jnp.float32),
    )(xp, W, deg)

    p = _build_hop_kernel()(u1, src3, dst3, zrows)

    u2 = pl.pallas_call(
        _mid_body,
        grid=(_NBLK,),
        in_specs=[
            pl.BlockSpec((_BLK, C), lambda i: (i, 0)),
            pl.BlockSpec((_BLK, C), lambda i: (i, 0)),
            pl.BlockSpec((_BLK,), lambda i: (i,)),
        ],
        out_specs=pl.BlockSpec((_BLK, C), lambda i: (i, 0)),
        out_shape=jax.ShapeDtypeStruct((NP, C), jnp.float32),
    )(p, u1, deg)

    q = _build_hop_kernel()(u2, src3, dst3, zrows)

    out = pl.pallas_call(
        _fin_body,
        grid=(_NBLK,),
        in_specs=[
            pl.BlockSpec((_BLK, C), lambda i: (i, 0)),
            pl.BlockSpec((_BLK, C), lambda i: (i, 0)),
            pl.BlockSpec((_BLK,), lambda i: (i,)),
            pl.BlockSpec((C,), lambda i: (0,)),
        ],
        out_specs=pl.BlockSpec((_BLK, C), lambda i: (i, 0)),
        out_shape=jax.ShapeDtypeStruct((NP, C), jnp.float32),
    )(q, u2, deg, b)

    return out[:N]


# speedup vs baseline: 2.3115x; 1.0027x over previous
import functools

import jax
import jax.numpy as jnp
from jax import lax
from jax.experimental import pallas as pl
from jax.experimental.pallas import tpu as pltpu
from jax.experimental.pallas import tpu_sc as plsc

N = 10000
E = 160000
F_IN = 256
C = 128

NC = 2
NS = 16
NP = 10240
EB = 128
NB = 80
BLK8 = 8
NBLK8 = NB // BLK8
EPT = NB * EB
EP = NS * EPT
ROWS_PER_TILE = NP // NS


def _mesh():
    return plsc.VectorSubcoreMesh(
        core_axis_name="c", subcore_axis_name="s", num_cores=NC, num_subcores=NS
    )


@functools.cache
def _build_deg_kernel():
    @functools.partial(
        pl.kernel,
        out_type=jax.ShapeDtypeStruct((NP,), jnp.float32),
        mesh=_mesh(),
        scratch_types=[
            pltpu.VMEM((NB, EB), jnp.int32),
            pltpu.VMEM((EB,), jnp.float32),
            pltpu.VMEM_SHARED((NP,), jnp.float32),
        ],
    )
    def _deg_kernel(dst_hbm, zeros_hbm, deg_hbm, dst_all, ones_v, acc):
        c = lax.axis_index("c")
        s = lax.axis_index("s")

        @pl.when(c == 0)
        def _():
            for i in range(EB // 16):
                ones_v[pl.ds(i * 16, 16)] = jnp.full((16,), 1.0,
                                                     dtype=jnp.float32)
            pltpu.sync_copy(dst_hbm.at[s], dst_all)
            pltpu.sync_copy(
                zeros_hbm.at[pl.ds(s * ROWS_PER_TILE, ROWS_PER_TILE)],
                acc.at[pl.ds(s * ROWS_PER_TILE, ROWS_PER_TILE)],
            )

        plsc.subcore_barrier()

        @pl.when(c == 0)
        def _():
            def body(j, carry):
                pltpu.sync_copy(ones_v, acc.at[dst_all.at[j]], add=True)
                return carry

            lax.fori_loop(0, NB, body, 0)

        plsc.subcore_barrier()

        @pl.when(c == 0)
        def _():
            pltpu.sync_copy(
                acc.at[pl.ds(s * ROWS_PER_TILE, ROWS_PER_TILE)],
                deg_hbm.at[pl.ds(s * ROWS_PER_TILE, ROWS_PER_TILE)],
            )

    return _deg_kernel


@functools.cache
def _build_hop_kernel():
    @functools.partial(
        pl.kernel,
        out_type=jax.ShapeDtypeStruct((NP, C), jnp.float32),
        mesh=_mesh(),
        scratch_types=[
            pltpu.VMEM((NB, EB), jnp.int32),
            pltpu.VMEM((2, BLK8, EB), jnp.int32),
            pltpu.VMEM((EB, C), jnp.float32),
            pltpu.VMEM((EB, C), jnp.float32),
            pltpu.VMEM_SHARED((NP, C), jnp.float32),
            pltpu.SemaphoreType.DMA,
            pltpu.SemaphoreType.DMA,
            pltpu.SemaphoreType.DMA,
            pltpu.SemaphoreType.DMA,
        ],
    )
    def _hop_kernel(u_hbm, src_hbm, dst_hbm, zrows_hbm, out_hbm,
                    src_all, dring, rows0, rows1, acc,
                    sem0, sem1, dsem0, dsem1):
        rows = (rows0, rows1)
        sems = (sem0, sem1)
        dsems = (dsem0, dsem1)
        c = lax.axis_index("c")
        s = lax.axis_index("s")

        @pl.when(c == 0)
        def _():
            pltpu.sync_copy(src_hbm.at[s], src_all)
            pltpu.sync_copy(dst_hbm.at[s, pl.ds(0, BLK8)], dring.at[0])
            pltpu.sync_copy(
                zrows_hbm.at[pl.ds(s * ROWS_PER_TILE, ROWS_PER_TILE)],
                acc.at[pl.ds(s * ROWS_PER_TILE, ROWS_PER_TILE)],
            )

        plsc.subcore_barrier()

        @pl.when(c == 0)
        def _():
            pltpu.async_copy(dst_hbm.at[s, pl.ds(BLK8, BLK8)],
                             dring.at[1], dsems[1])
            pltpu.async_copy(u_hbm.at[src_all.at[0]], rows[0], sems[0])

            def block2(k2, carry):
              for h in range(2):
                k = k2 * 2 + h

                @pl.when(k > 0)
                def _():
                    pltpu.make_async_copy(
                        dst_hbm.at[s, pl.ds(k * BLK8, BLK8)],
                        dring.at[h], dsems[h]).wait()

                for slot in range(BLK8):
                    b = slot % 2
                    nxt = 1 - b
                    j = k * BLK8 + slot

                    @pl.when(j + 1 < NB)
                    def _():
                        for hh in range(2):
                            lo = hh * (EB // 2)
                            pltpu.async_copy(
                                u_hbm.at[src_all.at[j + 1, pl.ds(lo, EB // 2)]],
                                rows[nxt].at[pl.ds(lo, EB // 2)], sems[nxt])

                    pltpu.make_async_copy(u_hbm.at[src_all.at[j]],
                                          rows[b], sems[b]).wait()
                    pltpu.sync_copy(rows[b], acc.at[dring.at[h, slot]],
                                    add=True)

                @pl.when(k + 2 < NBLK8)
                def _():
                    pltpu.async_copy(
                        dst_hbm.at[s, pl.ds((k + 2) * BLK8, BLK8)],
                        dring.at[h], dsems[h])

              return carry

            lax.fori_loop(0, NBLK8 // 2, block2, 0)

        plsc.subcore_barrier()

        @pl.when(c == 0)
        def _():
            pltpu.sync_copy(
                acc.at[pl.ds(s * ROWS_PER_TILE, ROWS_PER_TILE)],
                out_hbm.at[pl.ds(s * ROWS_PER_TILE, ROWS_PER_TILE)],
            )

    return _hop_kernel


_BLK = 256
_NBLK = NP // _BLK


def _mm_body(x_ref, w_ref, deg_ref, u1_ref):
    dinv = lax.rsqrt(deg_ref[...] + 1.0)
    z = lax.dot_general(x_ref[...], w_ref[...],
                        (((1,), (1,)), ((), ())),
                        preferred_element_type=jnp.float32)
    u1_ref[...] = z * dinv[:, None]


def _mid_body(p_ref, u1_ref, deg_ref, u2_ref):
    dinv = lax.rsqrt(deg_ref[...] + 1.0)
    t = p_ref[...] + u1_ref[...]
    u2_ref[...] = t * (dinv * dinv)[:, None]


def _fin_body(q_ref, u2_ref, deg_ref, b_ref, o_ref):
    dinv = lax.rsqrt(deg_ref[...] + 1.0)
    t = q_ref[...] + u2_ref[...]
    logits = t * dinv[:, None] + b_ref[...][None, :]
    m = jnp.max(logits, axis=1, keepdims=True)
    sh = logits - m
    lse = jnp.log(jnp.sum(jnp.exp(sh), axis=1, keepdims=True))
    o_ref[...] = sh - lse


def kernel(x, edge_index, W, b):
    x = x.astype(jnp.float32)
    W = W.astype(jnp.float32)
    b = b.astype(jnp.float32)
    src = edge_index[0].astype(jnp.int32)
    dst = edge_index[1].astype(jnp.int32)

    xp = jnp.pad(x, ((0, NP - N), (0, 0)))
    pad_e = EP - E
    spread = N + (jnp.arange(pad_e, dtype=jnp.int32) % (NP - N))
    src_p = jnp.concatenate([src, spread])
    dst_p = jnp.concatenate([dst, spread])
    src3 = src_p.reshape(NS, NB, EB)
    dst3 = dst_p.reshape(NS, NB, EB)

    zeros1 = jnp.zeros((NP,), jnp.float32)
    zrows = jnp.zeros((NP, C), jnp.float32)

    deg = _build_deg_kernel()(dst3, zeros1)

    u1 = pl.pallas_call(
        _mm_body,
        grid=(_NBLK,),
        in_specs=[
            pl.BlockSpec((_BLK, F_IN), lambda i: (i, 0)),
            pl.BlockSpec((C, F_IN), lambda i: (0, 0)),
            pl.BlockSpec((_BLK,), lambda i: (i,)),
        ],
        out_specs=pl.BlockSpec((_BLK, C), lambda i: (i, 0)),
        out_shape=jax.ShapeDtypeStruct((NP, C), jnp.float32),
    )(xp, W, deg)

    p = _build_hop_kernel()(u1, src3, dst3, zrows)

    u2 = pl.pallas_call(
        _mid_body,
        grid=(_NBLK,),
        in_specs=[
            pl.BlockSpec((_BLK, C), lambda i: (i, 0)),
            pl.BlockSpec((_BLK, C), lambda i: (i, 0)),
            pl.BlockSpec((_BLK,), lambda i: (i,)),
        ],
        out_specs=pl.BlockSpec((_BLK, C), lambda i: (i, 0)),
        out_shape=jax.ShapeDtypeStruct((NP, C), jnp.float32),
    )(p, u1, deg)

    q = _build_hop_kernel()(u2, src3, dst3, zrows)

    out = pl.pallas_call(
        _fin_body,
        grid=(_NBLK,),
        in_specs=[
            pl.BlockSpec((_BLK, C), lambda i: (i, 0)),
            pl.BlockSpec((_BLK, C), lambda i: (i, 0)),
            pl.BlockSpec((_BLK,), lambda i: (i,)),
            pl.BlockSpec((C,), lambda i: (0,)),
        ],
        out_specs=pl.BlockSpec((_BLK, C), lambda i: (i, 0)),
        out_shape=jax.ShapeDtypeStruct((NP, C), jnp.float32),
    )(q, u2, deg, b)

    return out[:N]


# speedup vs baseline: 2.3156x; 1.0017x over previous
import functools

import jax
import jax.numpy as jnp
from jax import lax
from jax.experimental import pallas as pl
from jax.experimental.pallas import tpu as pltpu
from jax.experimental.pallas import tpu_sc as plsc

N = 10000
E = 160000
F_IN = 256
C = 128

NC = 2
NS = 16
NP = 10240
EB = 128
NB = 80
BLK8 = 8
NBLK8 = NB // BLK8
EPT = NB * EB
EP = NS * EPT
ROWS_PER_TILE = NP // NS


def _mesh():
    return plsc.VectorSubcoreMesh(
        core_axis_name="c", subcore_axis_name="s", num_cores=NC, num_subcores=NS
    )


@functools.cache
def _build_deg_kernel():
    @functools.partial(
        pl.kernel,
        out_type=jax.ShapeDtypeStruct((NP,), jnp.float32),
        mesh=_mesh(),
        scratch_types=[
            pltpu.VMEM((NB, EB), jnp.int32),
            pltpu.VMEM((EB,), jnp.float32),
            pltpu.VMEM_SHARED((NP,), jnp.float32),
        ],
    )
    def _deg_kernel(dst_hbm, zeros_hbm, deg_hbm, dst_all, ones_v, acc):
        c = lax.axis_index("c")
        s = lax.axis_index("s")

        @pl.when(c == 0)
        def _():
            for i in range(EB // 16):
                ones_v[pl.ds(i * 16, 16)] = jnp.full((16,), 1.0,
                                                     dtype=jnp.float32)
            pltpu.sync_copy(dst_hbm.at[s], dst_all)
            pltpu.sync_copy(
                zeros_hbm.at[pl.ds(s * ROWS_PER_TILE, ROWS_PER_TILE)],
                acc.at[pl.ds(s * ROWS_PER_TILE, ROWS_PER_TILE)],
            )

        plsc.subcore_barrier()

        @pl.when(c == 0)
        def _():
            def body(j, carry):
                pltpu.sync_copy(ones_v, acc.at[dst_all.at[j]], add=True)
                return carry

            lax.fori_loop(0, NB, body, 0)

        plsc.subcore_barrier()

        @pl.when(c == 0)
        def _():
            pltpu.sync_copy(
                acc.at[pl.ds(s * ROWS_PER_TILE, ROWS_PER_TILE)],
                deg_hbm.at[pl.ds(s * ROWS_PER_TILE, ROWS_PER_TILE)],
            )

    return _deg_kernel


@functools.cache
def _build_hop_kernel():
    @functools.partial(
        pl.kernel,
        out_type=jax.ShapeDtypeStruct((NP, C), jnp.float32),
        mesh=_mesh(),
        scratch_types=[
            pltpu.VMEM((NB, EB), jnp.int32),
            pltpu.VMEM((2, BLK8, EB), jnp.int32),
            pltpu.VMEM((EB, C), jnp.float32),
            pltpu.VMEM((EB, C), jnp.float32),
            pltpu.VMEM_SHARED((NP, C), jnp.float32),
            pltpu.SemaphoreType.DMA,
            pltpu.SemaphoreType.DMA,
            pltpu.SemaphoreType.DMA,
            pltpu.SemaphoreType.DMA,
            pltpu.SemaphoreType.DMA,
            pltpu.SemaphoreType.DMA,
        ],
    )
    def _hop_kernel(u_hbm, src_hbm, dst_hbm, zrows_hbm, out_hbm,
                    src_all, dring, rows0, rows1, acc,
                    sem0, sem1, dsem0, dsem1, ssem0, ssem1):
        rows = (rows0, rows1)
        sems = (sem0, sem1)
        dsems = (dsem0, dsem1)
        ssems = (ssem0, ssem1)
        c = lax.axis_index("c")
        s = lax.axis_index("s")

        @pl.when(c == 0)
        def _():
            pltpu.sync_copy(src_hbm.at[s], src_all)
            pltpu.sync_copy(dst_hbm.at[s, pl.ds(0, BLK8)], dring.at[0])
            pltpu.sync_copy(
                zrows_hbm.at[pl.ds(s * ROWS_PER_TILE, ROWS_PER_TILE)],
                acc.at[pl.ds(s * ROWS_PER_TILE, ROWS_PER_TILE)],
            )

        plsc.subcore_barrier()

        @pl.when(c == 0)
        def _():
            pltpu.async_copy(dst_hbm.at[s, pl.ds(BLK8, BLK8)],
                             dring.at[1], dsems[1])
            pltpu.async_copy(u_hbm.at[src_all.at[0]], rows[0], sems[0])

            def block2(k2, carry):
              for h in range(2):
                k = k2 * 2 + h

                @pl.when(k > 0)
                def _():
                    pltpu.make_async_copy(
                        dst_hbm.at[s, pl.ds(k * BLK8, BLK8)],
                        dring.at[h], dsems[h]).wait()

                for slot in range(BLK8):
                    b = slot % 2
                    nxt = 1 - b
                    j = k * BLK8 + slot

                    @pl.when(j + 1 < NB)
                    def _():
                        @pl.when(j >= 1)
                        def _():
                            pltpu.make_async_copy(
                                rows[nxt], acc.at[dring.at[h, slot]],
                                ssems[nxt]).wait()

                        for hh in range(2):
                            lo = hh * (EB // 2)
                            pltpu.async_copy(
                                u_hbm.at[src_all.at[j + 1, pl.ds(lo, EB // 2)]],
                                rows[nxt].at[pl.ds(lo, EB // 2)], sems[nxt])

                    pltpu.make_async_copy(u_hbm.at[src_all.at[j]],
                                          rows[b], sems[b]).wait()
                    pltpu.async_copy(rows[b], acc.at[dring.at[h, slot]],
                                     ssems[b], add=True)

                @pl.when(k + 2 < NBLK8)
                def _():
                    pltpu.async_copy(
                        dst_hbm.at[s, pl.ds((k + 2) * BLK8, BLK8)],
                        dring.at[h], dsems[h])

              return carry

            lax.fori_loop(0, NBLK8 // 2, block2, 0)
            for b in range(2):
                pltpu.make_async_copy(rows[b], acc.at[dring.at[1, 0]],
                                      ssems[b]).wait()

        plsc.subcore_barrier()

        @pl.when(c == 0)
        def _():
            pltpu.sync_copy(
                acc.at[pl.ds(s * ROWS_PER_TILE, ROWS_PER_TILE)],
                out_hbm.at[pl.ds(s * ROWS_PER_TILE, ROWS_PER_TILE)],
            )

    return _hop_kernel


_BLK = 256
_NBLK = NP // _BLK


def _mm_body(x_ref, w_ref, deg_ref, u1_ref):
    dinv = lax.rsqrt(deg_ref[...] + 1.0)
    z = lax.dot_general(x_ref[...], w_ref[...],
                        (((1,), (1,)), ((), ())),
                        preferred_element_type=jnp.float32)
    u1_ref[...] = z * dinv[:, None]


def _mid_body(p_ref, u1_ref, deg_ref, u2_ref):
    dinv = lax.rsqrt(deg_ref[...] + 1.0)
    t = p_ref[...] + u1_ref[...]
    u2_ref[...] = t * (dinv * dinv)[:, None]


def _fin_body(q_ref, u2_ref, deg_ref, b_ref, o_ref):
    dinv = lax.rsqrt(deg_ref[...] + 1.0)
    t = q_ref[...] + u2_ref[...]
    logits = t * dinv[:, None] + b_ref[...][None, :]
    m = jnp.max(logits, axis=1, keepdims=True)
    sh = logits - m
    lse = jnp.log(jnp.sum(jnp.exp(sh), axis=1, keepdims=True))
    o_ref[...] = sh - lse


def kernel(x, edge_index, W, b):
    x = x.astype(jnp.float32)
    W = W.astype(jnp.float32)
    b = b.astype(jnp.float32)
    src = edge_index[0].astype(jnp.int32)
    dst = edge_index[1].astype(jnp.int32)

    xp = jnp.pad(x, ((0, NP - N), (0, 0)))
    pad_e = EP - E
    spread = N + (jnp.arange(pad_e, dtype=jnp.int32) % (NP - N))
    src_p = jnp.concatenate([src, spread])
    dst_p = jnp.concatenate([dst, spread])
    src3 = src_p.reshape(NS, NB, EB)
    dst3 = dst_p.reshape(NS, NB, EB)

    zeros1 = jnp.zeros((NP,), jnp.float32)
    zrows = jnp.zeros((NP, C), jnp.float32)

    deg = _build_deg_kernel()(dst3, zeros1)

    u1 = pl.pallas_call(
        _mm_body,
        grid=(_NBLK,),
        in_specs=[
            pl.BlockSpec((_BLK, F_IN), lambda i: (i, 0)),
            pl.BlockSpec((C, F_IN), lambda i: (0, 0)),
            pl.BlockSpec((_BLK,), lambda i: (i,)),
        ],
        out_specs=pl.BlockSpec((_BLK, C), lambda i: (i, 0)),
        out_shape=jax.ShapeDtypeStruct((NP, C), jnp.float32),
    )(xp, W, deg)

    p = _build_hop_kernel()(u1, src3, dst3, zrows)

    u2 = pl.pallas_call(
        _mid_body,
        grid=(_NBLK,),
        in_specs=[
            pl.BlockSpec((_BLK, C), lambda i: (i, 0)),
            pl.BlockSpec((_BLK, C), lambda i: (i, 0)),
            pl.BlockSpec((_BLK,), lambda i: (i,)),
        ],
        out_specs=pl.BlockSpec((_BLK, C), lambda i: (i, 0)),
        out_shape=jax.ShapeDtypeStruct((NP, C), jnp.float32),
    )(p, u1, deg)

    q = _build_hop_kernel()(u2, src3, dst3, zrows)

    out = pl.pallas_call(
        _fin_body,
        grid=(_NBLK,),
        in_specs=[
            pl.BlockSpec((_BLK, C), lambda i: (i, 0)),
            pl.BlockSpec((_BLK, C), lambda i: (i, 0)),
            pl.BlockSpec((_BLK,), lambda i: (i,)),
            pl.BlockSpec((C,), lambda i: (0,)),
        ],
        out_specs=pl.BlockSpec((_BLK, C), lambda i: (i, 0)),
        out_shape=jax.ShapeDtypeStruct((NP, C), jnp.float32),
    )(q, u2, deg, b)

    return out[:N]


# speedup vs baseline: 2.5480x; 1.1004x over previous
import functools

import jax
import jax.numpy as jnp
from jax import lax
from jax.experimental import pallas as pl
from jax.experimental.pallas import tpu as pltpu
from jax.experimental.pallas import tpu_sc as plsc

N = 10000
E = 160000
F_IN = 256
C = 128

NC = 2
NS = 16
NP = 10240
EB = 128
NB = 80
BLK8 = 8
NBLK8 = NB // BLK8
EPT = NB * EB
EP = NS * EPT
ROWS_PER_TILE = NP // NS


def _mesh():
    return plsc.VectorSubcoreMesh(
        core_axis_name="c", subcore_axis_name="s", num_cores=NC, num_subcores=NS
    )


@functools.cache
def _build_deg_kernel():
    @functools.partial(
        pl.kernel,
        out_type=jax.ShapeDtypeStruct((NP,), jnp.float32),
        mesh=_mesh(),
        scratch_types=[
            pltpu.VMEM((NB, EB), jnp.int32),
            pltpu.VMEM((EB,), jnp.float32),
            pltpu.VMEM_SHARED((NP,), jnp.float32),
        ],
    )
    def _deg_kernel(dst_hbm, zeros_hbm, deg_hbm, dst_all, ones_v, acc):
        c = lax.axis_index("c")
        s = lax.axis_index("s")

        @pl.when(c == 0)
        def _():
            for i in range(EB // 16):
                ones_v[pl.ds(i * 16, 16)] = jnp.full((16,), 1.0,
                                                     dtype=jnp.float32)
            pltpu.sync_copy(dst_hbm.at[s], dst_all)
            pltpu.sync_copy(
                zeros_hbm.at[pl.ds(s * ROWS_PER_TILE, ROWS_PER_TILE)],
                acc.at[pl.ds(s * ROWS_PER_TILE, ROWS_PER_TILE)],
            )

        plsc.subcore_barrier()

        @pl.when(c == 0)
        def _():
            def body(j, carry):
                pltpu.sync_copy(ones_v, acc.at[dst_all.at[j]], add=True)
                return carry

            lax.fori_loop(0, NB, body, 0)

        plsc.subcore_barrier()

        @pl.when(c == 0)
        def _():
            pltpu.sync_copy(
                acc.at[pl.ds(s * ROWS_PER_TILE, ROWS_PER_TILE)],
                deg_hbm.at[pl.ds(s * ROWS_PER_TILE, ROWS_PER_TILE)],
            )

    return _deg_kernel


@functools.cache
def _build_hop_kernel():
    @functools.partial(
        pl.kernel,
        out_type=jax.ShapeDtypeStruct((NP, C), jnp.float32),
        mesh=_mesh(),
        scratch_types=[
            pltpu.VMEM((NB, EB), jnp.int32),
            pltpu.VMEM((2, BLK8, EB), jnp.int32),
            pltpu.VMEM((EB, C), jnp.float32),
            pltpu.VMEM((EB, C), jnp.float32),
            pltpu.VMEM_SHARED((NP, C), jnp.float32),
            pltpu.SemaphoreType.DMA,
            pltpu.SemaphoreType.DMA,
            pltpu.SemaphoreType.DMA,
            pltpu.SemaphoreType.DMA,
            pltpu.SemaphoreType.DMA,
            pltpu.SemaphoreType.DMA,
        ],
    )
    def _hop_kernel(u_hbm, src_hbm, dst_hbm, zrows_hbm, out_hbm,
                    src_all, dring, rows0, rows1, acc,
                    sem0, sem1, dsem0, dsem1, ssem0, ssem1):
        rows = (rows0, rows1)
        sems = (sem0, sem1)
        dsems = (dsem0, dsem1)
        ssems = (ssem0, ssem1)
        c = lax.axis_index("c")
        s = lax.axis_index("s")

        @pl.when(c == 0)
        def _():
            pltpu.sync_copy(src_hbm.at[s], src_all)
            pltpu.sync_copy(dst_hbm.at[s, pl.ds(0, BLK8)], dring.at[0])
            pltpu.sync_copy(
                zrows_hbm.at[pl.ds(s * ROWS_PER_TILE, ROWS_PER_TILE)],
                acc.at[pl.ds(s * ROWS_PER_TILE, ROWS_PER_TILE)],
            )

        plsc.subcore_barrier()

        @pl.when(c == 0)
        def _():
            pltpu.async_copy(dst_hbm.at[s, pl.ds(BLK8, BLK8)],
                             dring.at[1], dsems[1])
            pltpu.async_copy(u_hbm.at[src_all.at[0]], rows[0], sems[0])

            def block2(k2, carry):
              for h in range(2):
                k = k2 * 2 + h

                @pl.when(k > 0)
                def _():
                    pltpu.make_async_copy(
                        dst_hbm.at[s, pl.ds(k * BLK8, BLK8)],
                        dring.at[h], dsems[h]).wait()

                for slot in range(BLK8):
                    b = slot % 2
                    nxt = 1 - b
                    j = k * BLK8 + slot

                    @pl.when(j + 1 < NB)
                    def _():
                        @pl.when(j >= 1)
                        def _():
                            pltpu.make_async_copy(
                                rows[nxt], acc.at[dring.at[h, slot]],
                                ssems[nxt]).wait()

                        for hh in range(2):
                            lo = hh * (EB // 2)
                            pltpu.async_copy(
                                u_hbm.at[src_all.at[j + 1, pl.ds(lo, EB // 2)]],
                                rows[nxt].at[pl.ds(lo, EB // 2)], sems[nxt])

                    pltpu.make_async_copy(u_hbm.at[src_all.at[j]],
                                          rows[b], sems[b]).wait()
                    pltpu.async_copy(rows[b], acc.at[dring.at[h, slot]],
                                     ssems[b], add=True)

                @pl.when(k + 2 < NBLK8)
                def _():
                    pltpu.async_copy(
                        dst_hbm.at[s, pl.ds((k + 2) * BLK8, BLK8)],
                        dring.at[h], dsems[h])

              return carry

            lax.fori_loop(0, NBLK8 // 2, block2, 0)
            for b in range(2):
                pltpu.make_async_copy(rows[b], acc.at[dring.at[1, 0]],
                                      ssems[b]).wait()

        plsc.subcore_barrier()

        @pl.when(c == 0)
        def _():
            pltpu.sync_copy(
                acc.at[pl.ds(s * ROWS_PER_TILE, ROWS_PER_TILE)],
                out_hbm.at[pl.ds(s * ROWS_PER_TILE, ROWS_PER_TILE)],
            )

    return _hop_kernel


_BLK = 512
_NBLK = NP // _BLK


def _mm_body(x_ref, w_ref, deg_ref, u1_ref):
    dinv = lax.rsqrt(deg_ref[...] + 1.0)
    z = lax.dot_general(x_ref[...], w_ref[...],
                        (((1,), (1,)), ((), ())),
                        preferred_element_type=jnp.float32)
    u1_ref[...] = z * dinv[:, None]


def _mid_body(p_ref, u1_ref, deg_ref, u2_ref):
    dinv = lax.rsqrt(deg_ref[...] + 1.0)
    t = p_ref[...] + u1_ref[...]
    u2_ref[...] = t * (dinv * dinv)[:, None]


def _fin_body(q_ref, u2_ref, deg_ref, b_ref, o_ref):
    dinv = lax.rsqrt(deg_ref[...] + 1.0)
    t = q_ref[...] + u2_ref[...]
    logits = t * dinv[:, None] + b_ref[...][None, :]
    m = jnp.max(logits, axis=1, keepdims=True)
    sh = logits - m
    lse = jnp.log(jnp.sum(jnp.exp(sh), axis=1, keepdims=True))
    o_ref[...] = sh - lse


def kernel(x, edge_index, W, b):
    x = x.astype(jnp.float32)
    W = W.astype(jnp.float32)
    b = b.astype(jnp.float32)
    src = edge_index[0].astype(jnp.int32)
    dst = edge_index[1].astype(jnp.int32)

    xp = jnp.pad(x, ((0, NP - N), (0, 0)))
    pad_e = EP - E
    spread = N + (jnp.arange(pad_e, dtype=jnp.int32) % (NP - N))
    src_p = jnp.concatenate([src, spread])
    dst_p = jnp.concatenate([dst, spread])
    src3 = src_p.reshape(NS, NB, EB)
    dst3 = dst_p.reshape(NS, NB, EB)

    zeros1 = jnp.zeros((NP,), jnp.float32)
    zrows = jnp.zeros((NP, C), jnp.float32)

    deg = _build_deg_kernel()(dst3, zeros1)

    u1 = pl.pallas_call(
        _mm_body,
        grid=(_NBLK,),
        in_specs=[
            pl.BlockSpec((_BLK, F_IN), lambda i: (i, 0)),
            pl.BlockSpec((C, F_IN), lambda i: (0, 0)),
            pl.BlockSpec((_BLK,), lambda i: (i,)),
        ],
        out_specs=pl.BlockSpec((_BLK, C), lambda i: (i, 0)),
        out_shape=jax.ShapeDtypeStruct((NP, C), jnp.float32),
    )(xp, W, deg)

    p = _build_hop_kernel()(u1, src3, dst3, zrows)

    u2 = pl.pallas_call(
        _mid_body,
        grid=(_NBLK,),
        in_specs=[
            pl.BlockSpec((_BLK, C), lambda i: (i, 0)),
            pl.BlockSpec((_BLK, C), lambda i: (i, 0)),
            pl.BlockSpec((_BLK,), lambda i: (i,)),
        ],
        out_specs=pl.BlockSpec((_BLK, C), lambda i: (i, 0)),
        out_shape=jax.ShapeDtypeStruct((NP, C), jnp.float32),
    )(p, u1, deg)

    q = _build_hop_kernel()(u2, src3, dst3, zrows)

    out = pl.pallas_call(
        _fin_body,
        grid=(_NBLK,),
        in_specs=[
            pl.BlockSpec((_BLK, C), lambda i: (i, 0)),
            pl.BlockSpec((_BLK, C), lambda i: (i, 0)),
            pl.BlockSpec((_BLK,), lambda i: (i,)),
            pl.BlockSpec((C,), lambda i: (0,)),
        ],
        out_specs=pl.BlockSpec((_BLK, C), lambda i: (i, 0)),
        out_shape=jax.ShapeDtypeStruct((NP, C), jnp.float32),
    )(q, u2, deg, b)

    return out[:N]


# speedup vs baseline: 2.6745x; 1.0496x over previous
import functools

import jax
import jax.numpy as jnp
from jax import lax
from jax.experimental import pallas as pl
from jax.experimental.pallas import tpu as pltpu
from jax.experimental.pallas import tpu_sc as plsc

N = 10000
E = 160000
F_IN = 256
C = 128

NC = 2
NS = 16
NP = 10240
EB = 128
NB = 80
BLK8 = 8
NBLK8 = NB // BLK8
EPT = NB * EB
EP = NS * EPT
ROWS_PER_TILE = NP // NS


def _mesh():
    return plsc.VectorSubcoreMesh(
        core_axis_name="c", subcore_axis_name="s", num_cores=NC, num_subcores=NS
    )


@functools.cache
def _build_deg_kernel():
    @functools.partial(
        pl.kernel,
        out_type=jax.ShapeDtypeStruct((NP,), jnp.float32),
        mesh=_mesh(),
        scratch_types=[
            pltpu.VMEM((NB, EB), jnp.int32),
            pltpu.VMEM((EB,), jnp.float32),
            pltpu.VMEM_SHARED((NP,), jnp.float32),
        ],
    )
    def _deg_kernel(dst_hbm, zeros_hbm, deg_hbm, dst_all, ones_v, acc):
        c = lax.axis_index("c")
        s = lax.axis_index("s")

        @pl.when(c == 0)
        def _():
            for i in range(EB // 16):
                ones_v[pl.ds(i * 16, 16)] = jnp.full((16,), 1.0,
                                                     dtype=jnp.float32)
            pltpu.sync_copy(dst_hbm.at[s], dst_all)
            pltpu.sync_copy(
                zeros_hbm.at[pl.ds(s * ROWS_PER_TILE, ROWS_PER_TILE)],
                acc.at[pl.ds(s * ROWS_PER_TILE, ROWS_PER_TILE)],
            )

        plsc.subcore_barrier()

        @pl.when(c == 0)
        def _():
            def body(j, carry):
                pltpu.sync_copy(ones_v, acc.at[dst_all.at[j]], add=True)
                return carry

            lax.fori_loop(0, NB, body, 0)

        plsc.subcore_barrier()

        @pl.when(c == 0)
        def _():
            pltpu.sync_copy(
                acc.at[pl.ds(s * ROWS_PER_TILE, ROWS_PER_TILE)],
                deg_hbm.at[pl.ds(s * ROWS_PER_TILE, ROWS_PER_TILE)],
            )

    return _deg_kernel


@functools.cache
def _build_hop_kernel():
    @functools.partial(
        pl.kernel,
        out_type=jax.ShapeDtypeStruct((NP, C), jnp.float32),
        mesh=_mesh(),
        scratch_types=[
            pltpu.VMEM((NB, EB), jnp.int32),
            pltpu.VMEM((2, BLK8, EB), jnp.int32),
            pltpu.VMEM((EB, C), jnp.float32),
            pltpu.VMEM((EB, C), jnp.float32),
            pltpu.VMEM_SHARED((NP, C), jnp.float32),
            pltpu.SemaphoreType.DMA,
            pltpu.SemaphoreType.DMA,
            pltpu.SemaphoreType.DMA,
            pltpu.SemaphoreType.DMA,
            pltpu.SemaphoreType.DMA,
            pltpu.SemaphoreType.DMA,
        ],
    )
    def _hop_kernel(u_hbm, src_hbm, dst_hbm, zrows_hbm, out_hbm,
                    src_all, dring, rows0, rows1, acc,
                    sem0, sem1, dsem0, dsem1, ssem0, ssem1):
        rows = (rows0, rows1)
        sems = (sem0, sem1)
        dsems = (dsem0, dsem1)
        ssems = (ssem0, ssem1)
        c = lax.axis_index("c")
        s = lax.axis_index("s")

        @pl.when(c == 0)
        def _():
            pltpu.sync_copy(src_hbm.at[s], src_all)
            pltpu.sync_copy(dst_hbm.at[s, pl.ds(0, BLK8)], dring.at[0])
            pltpu.sync_copy(
                zrows_hbm.at[pl.ds(s * ROWS_PER_TILE, ROWS_PER_TILE)],
                acc.at[pl.ds(s * ROWS_PER_TILE, ROWS_PER_TILE)],
            )

        plsc.subcore_barrier()

        @pl.when(c == 0)
        def _():
            pltpu.async_copy(dst_hbm.at[s, pl.ds(BLK8, BLK8)],
                             dring.at[1], dsems[1])
            pltpu.async_copy(u_hbm.at[src_all.at[0]], rows[0], sems[0])

            def block2(k2, carry):
              for h in range(2):
                k = k2 * 2 + h

                @pl.when(k > 0)
                def _():
                    pltpu.make_async_copy(
                        dst_hbm.at[s, pl.ds(k * BLK8, BLK8)],
                        dring.at[h], dsems[h]).wait()

                for slot in range(BLK8):
                    b = slot % 2
                    nxt = 1 - b
                    j = k * BLK8 + slot

                    @pl.when(j + 1 < NB)
                    def _():
                        @pl.when(j >= 1)
                        def _():
                            pltpu.make_async_copy(
                                rows[nxt], acc.at[dring.at[h, slot]],
                                ssems[nxt]).wait()

                        for hh in range(2):
                            lo = hh * (EB // 2)
                            pltpu.async_copy(
                                u_hbm.at[src_all.at[j + 1, pl.ds(lo, EB // 2)]],
                                rows[nxt].at[pl.ds(lo, EB // 2)], sems[nxt])

                    pltpu.make_async_copy(u_hbm.at[src_all.at[j]],
                                          rows[b], sems[b]).wait()
                    pltpu.async_copy(rows[b], acc.at[dring.at[h, slot]],
                                     ssems[b], add=True)

                @pl.when(k + 2 < NBLK8)
                def _():
                    pltpu.async_copy(
                        dst_hbm.at[s, pl.ds((k + 2) * BLK8, BLK8)],
                        dring.at[h], dsems[h])

              return carry

            lax.fori_loop(0, NBLK8 // 2, block2, 0)
            for b in range(2):
                pltpu.make_async_copy(rows[b], acc.at[dring.at[1, 0]],
                                      ssems[b]).wait()

        plsc.subcore_barrier()

        @pl.when(c == 0)
        def _():
            pltpu.sync_copy(
                acc.at[pl.ds(s * ROWS_PER_TILE, ROWS_PER_TILE)],
                out_hbm.at[pl.ds(s * ROWS_PER_TILE, ROWS_PER_TILE)],
            )

    return _hop_kernel


_BLK = 1024
_NBLK = NP // _BLK


def _mm_body(x_ref, w_ref, deg_ref, u1_ref):
    dinv = lax.rsqrt(deg_ref[...] + 1.0)
    z = lax.dot_general(x_ref[...], w_ref[...],
                        (((1,), (1,)), ((), ())),
                        preferred_element_type=jnp.float32)
    u1_ref[...] = z * dinv[:, None]


def _mid_body(p_ref, u1_ref, deg_ref, u2_ref):
    dinv = lax.rsqrt(deg_ref[...] + 1.0)
    t = p_ref[...] + u1_ref[...]
    u2_ref[...] = t * (dinv * dinv)[:, None]


def _fin_body(q_ref, u2_ref, deg_ref, b_ref, o_ref):
    dinv = lax.rsqrt(deg_ref[...] + 1.0)
    t = q_ref[...] + u2_ref[...]
    logits = t * dinv[:, None] + b_ref[...][None, :]
    m = jnp.max(logits, axis=1, keepdims=True)
    sh = logits - m
    lse = jnp.log(jnp.sum(jnp.exp(sh), axis=1, keepdims=True))
    o_ref[...] = sh - lse


def kernel(x, edge_index, W, b):
    x = x.astype(jnp.float32)
    W = W.astype(jnp.float32)
    b = b.astype(jnp.float32)
    src = edge_index[0].astype(jnp.int32)
    dst = edge_index[1].astype(jnp.int32)

    xp = jnp.pad(x, ((0, NP - N), (0, 0)))
    pad_e = EP - E
    spread = N + (jnp.arange(pad_e, dtype=jnp.int32) % (NP - N))
    src_p = jnp.concatenate([src, spread])
    dst_p = jnp.concatenate([dst, spread])
    src3 = src_p.reshape(NS, NB, EB)
    dst3 = dst_p.reshape(NS, NB, EB)

    zeros1 = jnp.zeros((NP,), jnp.float32)
    zrows = jnp.zeros((NP, C), jnp.float32)

    deg = _build_deg_kernel()(dst3, zeros1)

    u1 = pl.pallas_call(
        _mm_body,
        grid=(_NBLK,),
        in_specs=[
            pl.BlockSpec((_BLK, F_IN), lambda i: (i, 0)),
            pl.BlockSpec((C, F_IN), lambda i: (0, 0)),
            pl.BlockSpec((_BLK,), lambda i: (i,)),
        ],
        out_specs=pl.BlockSpec((_BLK, C), lambda i: (i, 0)),
        out_shape=jax.ShapeDtypeStruct((NP, C), jnp.float32),
    )(xp, W, deg)

    p = _build_hop_kernel()(u1, src3, dst3, zrows)

    u2 = pl.pallas_call(
        _mid_body,
        grid=(_NBLK,),
        in_specs=[
            pl.BlockSpec((_BLK, C), lambda i: (i, 0)),
            pl.BlockSpec((_BLK, C), lambda i: (i, 0)),
            pl.BlockSpec((_BLK,), lambda i: (i,)),
        ],
        out_specs=pl.BlockSpec((_BLK, C), lambda i: (i, 0)),
        out_shape=jax.ShapeDtypeStruct((NP, C), jnp.float32),
    )(p, u1, deg)

    q = _build_hop_kernel()(u2, src3, dst3, zrows)

    out = pl.pallas_call(
        _fin_body,
        grid=(_NBLK,),
        in_specs=[
            pl.BlockSpec((_BLK, C), lambda i: (i, 0)),
            pl.BlockSpec((_BLK, C), lambda i: (i, 0)),
            pl.BlockSpec((_BLK,), lambda i: (i,)),
            pl.BlockSpec((C,), lambda i: (0,)),
        ],
        out_specs=pl.BlockSpec((_BLK, C), lambda i: (i, 0)),
        out_shape=jax.ShapeDtypeStruct((NP, C), jnp.float32),
    )(q, u2, deg, b)

    return out[:N]
